# baseline (speedup 1.0000x reference)
import functools
import jax
import jax.numpy as jnp
from jax import lax
from jax.experimental import pallas as pl
from jax.experimental.pallas import tpu as pltpu
from jax.experimental.pallas import tpu_sc as plsc

_HIDDEN = 2048
_E = 64
_G = 8
_PG = _E // _G
_TK = 8
_TKG = 4
_SCALE = 2.5
_NEG = -1e30


def _mm_block(hs_ref, w_ref, b_ref, eb_ref, out_ref):
    hs = hs_ref[...]
    w = w_ref[...]
    logits = jax.lax.dot_general(
        w, hs, (((1,), (1,)), ((), ())),
        preferred_element_type=jnp.float32)
    out_ref[...] = jax.nn.sigmoid(logits + b_ref[...]) + eb_ref[...]


def _tc_scores(hidden_states, W, b, eb):
    t = hidden_states.shape[0]
    bt = 2048
    return pl.pallas_call(
        _mm_block,
        grid=(t // bt,),
        in_specs=[
            pl.BlockSpec((bt, _HIDDEN), lambda i: (i, 0)),
            pl.BlockSpec((_E, _HIDDEN), lambda i: (0, 0)),
            pl.BlockSpec((_E, 1), lambda i: (0, 0)),
            pl.BlockSpec((_E, 1), lambda i: (0, 0)),
        ],
        out_specs=pl.BlockSpec((_E, bt), lambda i: (0, i)),
        out_shape=jax.ShapeDtypeStruct((_E, t), jnp.float32),
        compiler_params=pltpu.CompilerParams(
            dimension_semantics=("parallel",)),
    )(hidden_states, W, b.reshape(_E, 1), eb.reshape(_E, 1))


def _sc_routing(sfc):
    e, t = sfc.shape
    info = plsc.get_sparse_core_info()
    nw = info.num_cores * info.num_subcores
    tpw = t // nw
    nb = tpw // 16
    mesh = plsc.VectorSubcoreMesh(core_axis_name="c", subcore_axis_name="s")

    @functools.partial(
        pl.kernel, mesh=mesh,
        out_type=[
            jax.ShapeDtypeStruct((_TK, t), jnp.int32),
            jax.ShapeDtypeStruct((_TK, t), jnp.float32),
        ],
        scratch_types=[
            pltpu.VMEM((_E, tpw), jnp.float32),
            pltpu.VMEM((_TK, tpw), jnp.int32),
            pltpu.VMEM((_TK, tpw), jnp.float32),
        ],
    )
    def k(sfc_hbm, idx_hbm, wgt_hbm, s_vm, idx_vm, wgt_vm):
        wid = lax.axis_index("s") * info.num_cores + lax.axis_index("c")
        base = wid * tpw
        pltpu.sync_copy(sfc_hbm.at[:, pl.ds(base, tpw)], s_vm)

        lane = lax.iota(jnp.int32, 16)
        neg = jnp.full((16,), _NEG, jnp.float32)

        def batch(j, _):
            col = j * 16
            s = [s_vm[ee, pl.ds(col, 16)] for ee in range(_E)]
            gs = []
            for g in range(_G):
                sg = s[g * _PG:(g + 1) * _PG]
                m1 = sg[0]
                i1 = jnp.zeros((16,), jnp.int32)
                for ee in range(1, _PG):
                    c = sg[ee] > m1
                    m1 = jnp.where(c, sg[ee], m1)
                    i1 = jnp.where(c, ee, i1)
                m2 = neg
                for ee in range(_PG):
                    c = jnp.logical_and(i1 != ee, sg[ee] > m2)
                    m2 = jnp.where(c, sg[ee], m2)
                gs.append(m1 + m2)
            gmask = [jnp.zeros((16,), jnp.bool_) for _ in range(_G)]
            sel = list(gs)
            for _k in range(_TKG):
                m = sel[0]
                gi = jnp.zeros((16,), jnp.int32)
                for g in range(1, _G):
                    c = sel[g] > m
                    m = jnp.where(c, sel[g], m)
                    gi = jnp.where(c, g, gi)
                for g in range(_G):
                    hit = gi == g
                    gmask[g] = jnp.logical_or(gmask[g], hit)
                    sel[g] = jnp.where(hit, _NEG, sel[g])
            ms = [jnp.where(gmask[ee // _PG], s[ee], _NEG) for ee in range(_E)]
            wsum = jnp.zeros((16,), jnp.float32)
            picks = []
            for _k in range(_TK):
                m = ms[0]
                ei = jnp.zeros((16,), jnp.int32)
                for ee in range(1, _E):
                    c = ms[ee] > m
                    m = jnp.where(c, ms[ee], m)
                    ei = jnp.where(c, ee, ei)
                picks.append((ei, m))
                wsum = wsum + m
                for ee in range(_E):
                    ms[ee] = jnp.where(ei == ee, _NEG, ms[ee])
            scale = _SCALE / (wsum + 1e-20)
            for _k, (ei, m) in enumerate(picks):
                idx_vm[_k, pl.ds(col, 16)] = ei
                wgt_vm[_k, pl.ds(col, 16)] = m * scale
            return 0

        lax.fori_loop(0, nb, batch, 0)
        pltpu.sync_copy(idx_vm, idx_hbm.at[:, pl.ds(base, tpw)])
        pltpu.sync_copy(wgt_vm, wgt_hbm.at[:, pl.ds(base, tpw)])

    return k(sfc)


def kernel(hidden_states, W, b, e_score_correction_bias):
    t = hidden_states.shape[0]
    sfc = _tc_scores(hidden_states, W, b, e_score_correction_bias)
    idx, wgt = _sc_routing(sfc)
    return idx.T, wgt.T


# speedup vs baseline: 2.6945x; 2.6945x over previous
import jax
import jax.numpy as jnp
from jax.experimental import pallas as pl
from jax.experimental.pallas import tpu as pltpu

_HIDDEN = 2048
_E = 64
_G = 8
_PG = _E // _G
_TK = 8
_TKG = 4
_SCALE = 2.5
_NEG = -1e30


def _router_block(hs_ref, w_ref, b_ref, eb_ref, idx_ref, wgt_ref):
    hs = hs_ref[...]
    w = w_ref[...]
    bt = hs.shape[0]
    logits = jax.lax.dot_general(
        w, hs, (((1,), (1,)), ((), ())),
        preferred_element_type=jnp.float32)
    logits = logits + b_ref[...]
    scores = jax.nn.sigmoid(logits)
    sfc = scores + eb_ref[...]

    lane8 = jax.lax.broadcasted_iota(jnp.int32, (_PG, bt), 0)
    gparts = []
    for g in range(_G):
        s = sfc[g * _PG:(g + 1) * _PG, :]
        m1 = jnp.max(s, axis=0, keepdims=True)
        first = jnp.min(jnp.where(s == m1, lane8, _PG), axis=0, keepdims=True)
        m2 = jnp.max(jnp.where(lane8 == first, _NEG, s), axis=0, keepdims=True)
        gparts.append(m1 + m2)
    gs = jnp.concatenate(gparts, axis=0)

    gi = jax.lax.broadcasted_iota(jnp.int32, (_G, bt), 0)
    sel = gs
    gmask = jnp.zeros((_G, bt), jnp.bool_)
    for _ in range(_TKG):
        m = jnp.max(sel, axis=0, keepdims=True)
        first = jnp.min(jnp.where(sel == m, gi, _G), axis=0, keepdims=True)
        hit = gi == first
        gmask = jnp.logical_or(gmask, hit)
        sel = jnp.where(hit, _NEG, sel)

    parts = [jnp.where(gmask[g:g + 1, :], sfc[g * _PG:(g + 1) * _PG, :], _NEG)
             for g in range(_G)]
    ms = jnp.concatenate(parts, axis=0)
    ei = jax.lax.broadcasted_iota(jnp.int32, (_E, bt), 0)
    idxs, wgts = [], []
    for _ in range(_TK):
        m = jnp.max(ms, axis=0, keepdims=True)
        first = jnp.min(jnp.where(ms == m, ei, _E), axis=0, keepdims=True)
        idxs.append(first)
        wgts.append(m)
        ms = jnp.where(ei == first, _NEG, ms)
    idx = jnp.concatenate(idxs, axis=0)
    wgt = jnp.concatenate(wgts, axis=0)
    denom = jnp.sum(wgt, axis=0, keepdims=True) + 1e-20
    wgt = wgt * (_SCALE / denom)
    idx_ref[...] = idx
    wgt_ref[...] = wgt


def kernel(hidden_states, W, b, e_score_correction_bias):
    t = hidden_states.shape[0]
    bt = 2048
    grid = (t // bt,)
    b2 = b.reshape(_E, 1)
    eb2 = e_score_correction_bias.reshape(_E, 1)
    idx_t, wgt_t = pl.pallas_call(
        _router_block,
        grid=grid,
        in_specs=[
            pl.BlockSpec((bt, _HIDDEN), lambda i: (i, 0)),
            pl.BlockSpec((_E, _HIDDEN), lambda i: (0, 0)),
            pl.BlockSpec((_E, 1), lambda i: (0, 0)),
            pl.BlockSpec((_E, 1), lambda i: (0, 0)),
        ],
        out_specs=[
            pl.BlockSpec((_TK, bt), lambda i: (0, i)),
            pl.BlockSpec((_TK, bt), lambda i: (0, i)),
        ],
        out_shape=[
            jax.ShapeDtypeStruct((_TK, t), jnp.int32),
            jax.ShapeDtypeStruct((_TK, t), jnp.float32),
        ],
        compiler_params=pltpu.CompilerParams(
            dimension_semantics=("parallel",)),
    )(hidden_states, W, b2, eb2)
    return idx_t.T, wgt_t.T
